# Initial kernel scaffold; baseline (speedup 1.0000x reference)
import jax
import jax.numpy as jnp
from jax.experimental import pallas as pl


def kernel(H_l, HBar_l, edge_index_history, edge_index_sampled, W, b):
    raise NotImplementedError("write your pallas kernel here")


# speedup vs baseline: 3.0307x; 3.0307x over previous
import functools

import jax
import jax.numpy as jnp
from jax import lax
from jax.experimental import pallas as pl
from jax.experimental.pallas import tpu as pltpu
from jax.experimental.pallas import tpu_sc as plsc

N = 10000
E = 160000
D = 256
HALF = 128
NC = 2
NS = 16
NPAD = 10240
ROWS_PT = NPAD // NS
GSZ = 128
EPAD = 161792
GROUPS_PT = EPAD // (NS * GSZ)
PAD_DST = NPAD - 1


def _sc_agg_body(hbar2, hdelta2, src_h, dst_h, src_s, dst_s,
                 sum_h, sum_s, cnt_h, cnt_s,
                 accum, hist_part, rowsb, srcb, dstb, gidxb,
                 histb, redb, cntb, gsem):
    c = lax.axis_index("c")
    s = lax.axis_index("s")
    row0 = s * ROWS_PT
    zero16 = jnp.zeros((16,), jnp.float32)
    ones16 = jnp.ones((16,), jnp.float32)

    def _zh(i, carry):
        histb[pl.ds(i * 16, 16)] = zero16
        return carry
    lax.fori_loop(0, NPAD // 16, _zh, None)

    def _zero_accum():
        def _zb(r, carry):
            for k in range(HALF // 16):
                rowsb[r, pl.ds(k * 16, 16)] = zero16
            return carry
        lax.fori_loop(0, GSZ, _zb, None)
        for i in range(ROWS_PT // GSZ):
            pltpu.sync_copy(rowsb, accum.at[pl.ds(row0 + i * GSZ, GSZ)])

    def _do_pass(src_e, dst_e, feat2, count_flag):
        gbase = s * GROUPS_PT

        def body(g, carry):
            e0 = (gbase + g) * GSZ
            pltpu.sync_copy(src_e.at[pl.ds(e0, GSZ)], srcb)
            pltpu.sync_copy(dst_e.at[pl.ds(e0, GSZ)], dstb)
            for k in range(GSZ // 16):
                sv = srcb[pl.ds(k * 16, 16)]
                gidxb[pl.ds(k * 16, 16)] = sv * 2 + c
            pltpu.async_copy(feat2.at[gidxb], rowsb, gsem).wait()
            pltpu.sync_copy(rowsb, accum.at[dstb], add=True)

            @pl.when(count_flag)
            def _():
                for k in range(GSZ // 16):
                    dv = dstb[pl.ds(k * 16, 16)]
                    plsc.addupdate_scatter(histb, [dv], ones16)
            return carry

        lax.fori_loop(0, GROUPS_PT, body, None)

    def _drain(out_ref):
        for i in range(ROWS_PT // GSZ):
            r = row0 + i * GSZ
            pltpu.sync_copy(accum.at[pl.ds(r, GSZ)], rowsb)
            pltpu.sync_copy(rowsb, out_ref.at[pl.ds(r, GSZ), pl.ds(c * HALF, HALF)])

    _zero_accum()
    plsc.subcore_barrier()

    _do_pass(src_h, dst_h, hbar2, c == 0)
    plsc.subcore_barrier()
    _drain(sum_h)
    _zero_accum()
    plsc.subcore_barrier()

    _do_pass(src_s, dst_s, hdelta2, c == 1)
    plsc.subcore_barrier()
    _drain(sum_s)

    def _reduce_counts(out):
        pltpu.sync_copy(histb, hist_part.at[s])
        plsc.subcore_barrier()
        pltpu.sync_copy(hist_part.at[:, pl.ds(row0, ROWS_PT)], redb)

        def _red(k, carry):
            acc = zero16
            for j in range(NS):
                acc = acc + redb[j, pl.ds(k * 16, 16)]
            cntb[pl.ds(k * 16, 16)] = acc
            return carry
        lax.fori_loop(0, ROWS_PT // 16, _red, None)
        pltpu.sync_copy(cntb, out.at[pl.ds(row0, ROWS_PT)])

    @pl.when(c == 0)
    def _():
        _reduce_counts(cnt_h)

    @pl.when(c == 1)
    def _():
        _reduce_counts(cnt_s)


@functools.cache
def _get_sc_agg(interpret: bool = False):
    mesh = plsc.VectorSubcoreMesh(
        core_axis_name="c", subcore_axis_name="s", num_cores=NC, num_subcores=NS
    )
    return pl.kernel(
        _sc_agg_body,
        out_type=(
            jax.ShapeDtypeStruct((NPAD, D), jnp.float32),
            jax.ShapeDtypeStruct((NPAD, D), jnp.float32),
            jax.ShapeDtypeStruct((NPAD,), jnp.float32),
            jax.ShapeDtypeStruct((NPAD,), jnp.float32),
        ),
        mesh=mesh,
        scratch_types=[
            pltpu.VMEM_SHARED((NPAD, HALF), jnp.float32),
            pltpu.VMEM_SHARED((NS, NPAD), jnp.float32),
            pltpu.VMEM((GSZ, HALF), jnp.float32),
            pltpu.VMEM((GSZ,), jnp.int32),
            pltpu.VMEM((GSZ,), jnp.int32),
            pltpu.VMEM((GSZ,), jnp.int32),
            pltpu.VMEM((NPAD,), jnp.float32),
            pltpu.VMEM((NS, ROWS_PT), jnp.float32),
            pltpu.VMEM((ROWS_PT,), jnp.float32),
            pltpu.SemaphoreType.DMA,
        ],
        compiler_params=pltpu.CompilerParams(needs_layout_passes=False),
        interpret=interpret,
    )


_TCR = 1000


def _tc_pre_body(h_ref, hb_ref, w1_ref, b_ref, p_ref, hd_ref):
    hv = h_ref[...]
    hd_ref[...] = hv - hb_ref[...]
    p_ref[...] = (
        jnp.dot(hv, w1_ref[...], preferred_element_type=jnp.float32) + b_ref[...]
    )


def _tc_post_body(p_ref, sh0_ref, sh1_ref, ss0_ref, ss1_ref, ch_ref, cs_ref,
                  w2a_ref, w2b_ref, o_ref):
    rh = 1.0 / jnp.maximum(ch_ref[...], 1.0)
    rs = 1.0 / jnp.maximum(cs_ref[...], 1.0)
    hn0 = sh0_ref[...] * rh + ss0_ref[...] * rs
    hn1 = sh1_ref[...] * rh + ss1_ref[...] * rs
    acc = (
        p_ref[...]
        + jnp.dot(hn0, w2a_ref[...], preferred_element_type=jnp.float32)
        + jnp.dot(hn1, w2b_ref[...], preferred_element_type=jnp.float32)
    )
    o_ref[...] = jnp.maximum(acc, 0.0)


def _tc_pre(H_l, HBar_l, W1, b2):
    return pl.pallas_call(
        _tc_pre_body,
        grid=(N // _TCR,),
        in_specs=[
            pl.BlockSpec((_TCR, D), lambda i: (i, 0)),
            pl.BlockSpec((_TCR, D), lambda i: (i, 0)),
            pl.BlockSpec((D, D), lambda i: (0, 0)),
            pl.BlockSpec((1, D), lambda i: (0, 0)),
        ],
        out_specs=[
            pl.BlockSpec((_TCR, D), lambda i: (i, 0)),
            pl.BlockSpec((_TCR, D), lambda i: (i, 0)),
        ],
        out_shape=[
            jax.ShapeDtypeStruct((N, D), jnp.float32),
            jax.ShapeDtypeStruct((N, D), jnp.float32),
        ],
    )(H_l, HBar_l, W1, b2)


def _tc_post(P, sum_h, sum_s, cnt_h, cnt_s, W2a, W2b):
    return pl.pallas_call(
        _tc_post_body,
        grid=(N // _TCR,),
        in_specs=[
            pl.BlockSpec((_TCR, D), lambda i: (i, 0)),
            pl.BlockSpec((_TCR, HALF), lambda i: (i, 0)),
            pl.BlockSpec((_TCR, HALF), lambda i: (i, 1)),
            pl.BlockSpec((_TCR, HALF), lambda i: (i, 0)),
            pl.BlockSpec((_TCR, HALF), lambda i: (i, 1)),
            pl.BlockSpec((_TCR, 1), lambda i: (i, 0)),
            pl.BlockSpec((_TCR, 1), lambda i: (i, 0)),
            pl.BlockSpec((HALF, D), lambda i: (0, 0)),
            pl.BlockSpec((HALF, D), lambda i: (0, 0)),
        ],
        out_specs=pl.BlockSpec((_TCR, D), lambda i: (i, 0)),
        out_shape=jax.ShapeDtypeStruct((N, D), jnp.float32),
    )(P, sum_h, sum_h, sum_s, sum_s, cnt_h, cnt_s, W2a, W2b)


def kernel(H_l, HBar_l, edge_index_history, edge_index_sampled, W, b):
    src_h, dst_h = edge_index_history[0], edge_index_history[1]
    src_s, dst_s = edge_index_sampled[0], edge_index_sampled[1]

    pad = EPAD - E
    pad_src = jnp.zeros((pad,), jnp.int32)
    pad_dst = jnp.full((pad,), PAD_DST, jnp.int32)
    src_h = jnp.concatenate([src_h, pad_src])
    dst_h = jnp.concatenate([dst_h, pad_dst])
    src_s = jnp.concatenate([src_s, pad_src])
    dst_s = jnp.concatenate([dst_s, pad_dst])

    P, hdelta = _tc_pre(H_l, HBar_l, W[:D], b.reshape(1, D))

    hbar2 = HBar_l.reshape(2 * N, HALF)
    hdelta2 = hdelta.reshape(2 * N, HALF)

    sum_h, sum_s, cnt_h, cnt_s = _get_sc_agg()(
        hbar2, hdelta2, src_h, dst_h, src_s, dst_s
    )

    return _tc_post(
        P, sum_h, sum_s,
        cnt_h[:N].reshape(N, 1), cnt_s[:N].reshape(N, 1),
        W[D:D + HALF], W[D + HALF:],
    )


# speedup vs baseline: 3.2736x; 1.0801x over previous
import functools

import jax
import jax.numpy as jnp
from jax import lax
from jax.experimental import pallas as pl
from jax.experimental.pallas import tpu as pltpu
from jax.experimental.pallas import tpu_sc as plsc

N = 10000
E = 160000
D = 256
HALF = 128
NC = 2
NS = 16
NPAD = 10112
ROWS_PT = NPAD // NS
GSZ = 64
SCG = 8
NG = 160
NSC = NG // SCG
EROWS = NS * NG + SCG
EPAD = EROWS * GSZ
PAD_DST = NPAD - 1


def _sc_agg_body(hbar2, hdelta2, src_h, dst_h, src_s, dst_s,
                 sum_h, sum_s, cnt_h, cnt_s,
                 accum, hist_part, rowsA, rowsB,
                 srcA, dstA, gidxA, srcB, dstB, gidxB,
                 histb, redb, gsA, gsB, ssA, ssB):
    c = lax.axis_index("c")
    s = lax.axis_index("s")
    row0 = s * ROWS_PT
    zero16 = jnp.zeros((16,), jnp.float32)
    ones16 = jnp.ones((16,), jnp.float32)
    rows = (rowsA, rowsB)
    gsem = (gsA, gsB)
    ssem = (ssA, ssB)
    idxset = ((srcA, dstA, gidxA), (srcB, dstB, gidxB))

    def _zh(i, carry):
        histb[pl.ds(i * 16, 16)] = zero16
        return carry
    lax.fori_loop(0, NPAD // 16, _zh, None)

    _drain_chunks = [(i * GSZ, GSZ) for i in range(ROWS_PT // GSZ)]
    if ROWS_PT % GSZ:
        _drain_chunks.append(((ROWS_PT // GSZ) * GSZ, ROWS_PT % GSZ))

    def _zero_accum():
        def _zb(r, carry):
            for q in range(HALF // 16):
                rowsA[r, pl.ds(q * 16, 16)] = zero16
            return carry
        lax.fori_loop(0, GSZ, _zb, None)
        for off, ln in _drain_chunks:
            pltpu.sync_copy(rowsA.at[pl.ds(0, ln)], accum.at[pl.ds(row0 + off, ln)])

    def _drain(out_ref):
        for off, ln in _drain_chunks:
            pltpu.sync_copy(accum.at[pl.ds(row0 + off, ln)], rowsA.at[pl.ds(0, ln)])
            pltpu.sync_copy(rowsA.at[pl.ds(0, ln)],
                            out_ref.at[pl.ds(row0 + off, ln), pl.ds(c * HALF, HALF)])

    def _do_pass(src_e, dst_e, feat2, count_flag):
        def load_chunk(k, st):
            srcb, dstb, gidxb = idxset[st]
            r0 = s * NG + k * SCG
            pltpu.sync_copy(src_e.at[pl.ds(r0, SCG)], srcb)
            pltpu.sync_copy(dst_e.at[pl.ds(r0, SCG)], dstb)
            for j in range(SCG):
                for q in range(GSZ // 16):
                    gidxb[j, pl.ds(q * 16, 16)] = (
                        srcb[j, pl.ds(q * 16, 16)] * 2 + c)

        def start_gather(j, st, buf):
            gidxb = idxset[st][2]
            pltpu.async_copy(feat2.at[gidxb.at[j]], rows[buf], gsem[buf])

        def wait_gather(buf):
            pltpu.make_async_copy(feat2.at[gidxA.at[0]], rows[buf],
                                  gsem[buf]).wait()

        def start_scatter(j, st, buf):
            dstb = idxset[st][1]
            pltpu.async_copy(rows[buf], accum.at[dstb.at[j]], ssem[buf],
                             add=True)

        def wait_scatter(buf):
            pltpu.make_async_copy(rows[buf], accum.at[dstA.at[0]],
                                  ssem[buf]).wait()

        def hist_group(j, st):
            dstb = idxset[st][1]
            for q in range(GSZ // 16):
                dv = dstb[j, pl.ds(q * 16, 16)]
                plsc.addupdate_scatter(histb, [dv], ones16)

        def sixteen_groups(t, first, last):
            for j in range(16):
                st = j // 8
                buf = j % 2
                other = 1 - buf
                if not (first and j == 0):
                    wait_scatter(other)
                if j < 15:
                    start_gather((j + 1) % 8, (j + 1) // 8, other)
                else:
                    if last is None:
                        @pl.when(t < NSC // 2 - 1)
                        def _():
                            start_gather(0, 0, other)
                    elif not last:
                        start_gather(0, 0, other)
                wait_gather(buf)
                start_scatter(j % 8, st, buf)

                @pl.when(count_flag)
                def _():
                    hist_group(j % 8, st)

                if j == 5:
                    load_chunk(2 * t + 1, 1)
                if j == 13:
                    load_chunk(2 * t + 2, 0)

        load_chunk(0, 0)
        start_gather(0, 0, 0)
        sixteen_groups(0, True, NSC // 2 == 1)

        def body(t, carry):
            sixteen_groups(t, False, None)
            return carry
        lax.fori_loop(1, NSC // 2, body, None)

        wait_scatter(1)

    _zero_accum()
    plsc.subcore_barrier()

    _do_pass(src_h, dst_h, hbar2, c == 0)
    plsc.subcore_barrier()
    _drain(sum_h)
    _zero_accum()
    plsc.subcore_barrier()

    _do_pass(src_s, dst_s, hdelta2, c == 1)
    plsc.subcore_barrier()
    _drain(sum_s)

    def _reduce_counts(out):
        pltpu.sync_copy(histb, hist_part.at[s])
        plsc.subcore_barrier()
        nblk = jnp.where(s == NS - 1, NPAD // HALF - 5 * (NS - 1), 5)

        def blk(b, carry):
            col0 = pl.multiple_of((s * 5 + b) * HALF, HALF)
            pltpu.sync_copy(hist_part.at[:, pl.ds(col0, HALF)], redb)
            for q in range(HALF // 16):
                acc = zero16
                for j in range(NS):
                    acc = acc + redb[j, pl.ds(q * 16, 16)]
                rowsA[0, pl.ds(q * 16, 16)] = acc
            pltpu.sync_copy(rowsA.at[0], out.at[pl.ds(col0, HALF)])
            return carry
        lax.fori_loop(0, nblk, blk, None)

    @pl.when(c == 0)
    def _():
        _reduce_counts(cnt_h)

    @pl.when(c == 1)
    def _():
        _reduce_counts(cnt_s)


@functools.cache
def _get_sc_agg(interpret: bool = False):
    mesh = plsc.VectorSubcoreMesh(
        core_axis_name="c", subcore_axis_name="s", num_cores=NC, num_subcores=NS
    )
    return pl.kernel(
        _sc_agg_body,
        out_type=(
            jax.ShapeDtypeStruct((NPAD, D), jnp.float32),
            jax.ShapeDtypeStruct((NPAD, D), jnp.float32),
            jax.ShapeDtypeStruct((NPAD,), jnp.float32),
            jax.ShapeDtypeStruct((NPAD,), jnp.float32),
        ),
        mesh=mesh,
        scratch_types=[
            pltpu.VMEM_SHARED((NPAD, HALF), jnp.float32),
            pltpu.VMEM_SHARED((NS, NPAD), jnp.float32),
            pltpu.VMEM((GSZ, HALF), jnp.float32),
            pltpu.VMEM((GSZ, HALF), jnp.float32),
            pltpu.VMEM((SCG, GSZ), jnp.int32),
            pltpu.VMEM((SCG, GSZ), jnp.int32),
            pltpu.VMEM((SCG, GSZ), jnp.int32),
            pltpu.VMEM((SCG, GSZ), jnp.int32),
            pltpu.VMEM((SCG, GSZ), jnp.int32),
            pltpu.VMEM((SCG, GSZ), jnp.int32),
            pltpu.VMEM((NPAD,), jnp.float32),
            pltpu.VMEM((NS, HALF), jnp.float32),
            pltpu.SemaphoreType.DMA,
            pltpu.SemaphoreType.DMA,
            pltpu.SemaphoreType.DMA,
            pltpu.SemaphoreType.DMA,
        ],
        compiler_params=pltpu.CompilerParams(needs_layout_passes=False),
        interpret=interpret,
    )


_TCR = 1000


def _tc_pre_body(h_ref, hb_ref, w1_ref, b_ref, p_ref, hd_ref):
    hv = h_ref[...]
    hd_ref[...] = hv - hb_ref[...]
    p_ref[...] = (
        jnp.dot(hv, w1_ref[...], preferred_element_type=jnp.float32) + b_ref[...]
    )


def _tc_post_body(p_ref, sh0_ref, sh1_ref, ss0_ref, ss1_ref, ch_ref, cs_ref,
                  w2a_ref, w2b_ref, o_ref):
    rh = 1.0 / jnp.maximum(ch_ref[...], 1.0)
    rs = 1.0 / jnp.maximum(cs_ref[...], 1.0)
    hn0 = sh0_ref[...] * rh + ss0_ref[...] * rs
    hn1 = sh1_ref[...] * rh + ss1_ref[...] * rs
    acc = (
        p_ref[...]
        + jnp.dot(hn0, w2a_ref[...], preferred_element_type=jnp.float32)
        + jnp.dot(hn1, w2b_ref[...], preferred_element_type=jnp.float32)
    )
    o_ref[...] = jnp.maximum(acc, 0.0)


def _tc_pre(H_l, HBar_l, W1, b2):
    return pl.pallas_call(
        _tc_pre_body,
        grid=(N // _TCR,),
        in_specs=[
            pl.BlockSpec((_TCR, D), lambda i: (i, 0)),
            pl.BlockSpec((_TCR, D), lambda i: (i, 0)),
            pl.BlockSpec((D, D), lambda i: (0, 0)),
            pl.BlockSpec((1, D), lambda i: (0, 0)),
        ],
        out_specs=[
            pl.BlockSpec((_TCR, D), lambda i: (i, 0)),
            pl.BlockSpec((_TCR, D), lambda i: (i, 0)),
        ],
        out_shape=[
            jax.ShapeDtypeStruct((N, D), jnp.float32),
            jax.ShapeDtypeStruct((N, D), jnp.float32),
        ],
    )(H_l, HBar_l, W1, b2)


def _tc_post(P, sum_h, sum_s, cnt_h2, cnt_s2, W2a, W2b):
    return pl.pallas_call(
        _tc_post_body,
        grid=(N // _TCR,),
        in_specs=[
            pl.BlockSpec((_TCR, D), lambda i: (i, 0)),
            pl.BlockSpec((_TCR, HALF), lambda i: (i, 0)),
            pl.BlockSpec((_TCR, HALF), lambda i: (i, 1)),
            pl.BlockSpec((_TCR, HALF), lambda i: (i, 0)),
            pl.BlockSpec((_TCR, HALF), lambda i: (i, 1)),
            pl.BlockSpec((_TCR, 1), lambda i: (i, 0)),
            pl.BlockSpec((_TCR, 1), lambda i: (i, 0)),
            pl.BlockSpec((HALF, D), lambda i: (0, 0)),
            pl.BlockSpec((HALF, D), lambda i: (0, 0)),
        ],
        out_specs=pl.BlockSpec((_TCR, D), lambda i: (i, 0)),
        out_shape=jax.ShapeDtypeStruct((N, D), jnp.float32),
    )(P, sum_h, sum_h, sum_s, sum_s, cnt_h2, cnt_s2, W2a, W2b)


def kernel(H_l, HBar_l, edge_index_history, edge_index_sampled, W, b):
    src_h, dst_h = edge_index_history[0], edge_index_history[1]
    src_s, dst_s = edge_index_sampled[0], edge_index_sampled[1]

    pad = EPAD - E
    pad_src = jnp.zeros((pad,), jnp.int32)
    pad_dst = jnp.full((pad,), PAD_DST, jnp.int32)
    src_h = jnp.concatenate([src_h, pad_src]).reshape(EROWS, GSZ)
    dst_h = jnp.concatenate([dst_h, pad_dst]).reshape(EROWS, GSZ)
    src_s = jnp.concatenate([src_s, pad_src]).reshape(EROWS, GSZ)
    dst_s = jnp.concatenate([dst_s, pad_dst]).reshape(EROWS, GSZ)

    P, hdelta = _tc_pre(H_l, HBar_l, W[:D], b.reshape(1, D))

    hbar2 = HBar_l.reshape(2 * N, HALF)
    hdelta2 = hdelta.reshape(2 * N, HALF)

    sum_h, sum_s, cnt_h, cnt_s = _get_sc_agg()(
        hbar2, hdelta2, src_h, dst_h, src_s, dst_s
    )

    return _tc_post(
        P, sum_h, sum_s,
        cnt_h[:N].reshape(N, 1), cnt_s[:N].reshape(N, 1),
        W[D:D + HALF], W[D + HALF:],
    )


# speedup vs baseline: 3.3926x; 1.0363x over previous
import functools

import jax
import jax.numpy as jnp
from jax import lax
from jax.experimental import pallas as pl
from jax.experimental.pallas import tpu as pltpu
from jax.experimental.pallas import tpu_sc as plsc

N = 10000
E = 160000
D = 256
HALF = 128
NC = 2
NS = 16
NPAD = 10112
ROWS_PT = NPAD // NS
GSZ = 64
SCG = 8
NG = 160
NSC = NG // SCG
EROWS = NS * NG + SCG
EPAD = EROWS * GSZ
PAD_DST = NPAD - 1


def _sc_agg_body(hbar2, hdelta2, src_h, dst_h, src_s, dst_s,
                 sum_h, sum_s, cnt_h, cnt_s,
                 accum, hist_part, rowsA, rowsB,
                 srcA, dstA, gidxA, srcB, dstB, gidxB,
                 histb, redb, gsA, gsB, ssA, ssB, isA, isB):
    c = lax.axis_index("c")
    s = lax.axis_index("s")
    row0 = s * ROWS_PT
    zero16 = jnp.zeros((16,), jnp.float32)
    ones16 = jnp.ones((16,), jnp.float32)
    rows = (rowsA, rowsB)
    gsem = (gsA, gsB)
    ssem = (ssA, ssB)
    isem = (isA, isB)
    idxset = ((srcA, dstA, gidxA), (srcB, dstB, gidxB))

    def _zh(i, carry):
        histb[pl.ds(i * 16, 16)] = zero16
        return carry
    lax.fori_loop(0, NPAD // 16, _zh, None)

    _drain_chunks = [(i * GSZ, GSZ) for i in range(ROWS_PT // GSZ)]
    if ROWS_PT % GSZ:
        _drain_chunks.append(((ROWS_PT // GSZ) * GSZ, ROWS_PT % GSZ))

    def _zero_accum():
        def _zb(r, carry):
            for q in range(HALF // 16):
                rowsA[r, pl.ds(q * 16, 16)] = zero16
            return carry
        lax.fori_loop(0, GSZ, _zb, None)
        for off, ln in _drain_chunks:
            pltpu.async_copy(rowsA.at[pl.ds(0, ln)],
                             accum.at[pl.ds(row0 + off, ln)], gsA)
        for off, ln in _drain_chunks:
            pltpu.make_async_copy(rowsA.at[pl.ds(0, ln)],
                                  accum.at[pl.ds(row0 + off, ln)], gsA).wait()

    def _drain(out_ref):
        n = len(_drain_chunks)

        def d_in(i, buf):
            off, ln = _drain_chunks[i]
            pltpu.async_copy(accum.at[pl.ds(row0 + off, ln)],
                             rows[buf].at[pl.ds(0, ln)], gsem[buf])

        def d_in_wait(i, buf):
            off, ln = _drain_chunks[i]
            pltpu.make_async_copy(accum.at[pl.ds(row0 + off, ln)],
                                  rows[buf].at[pl.ds(0, ln)], gsem[buf]).wait()

        def d_out(i, buf):
            off, ln = _drain_chunks[i]
            pltpu.async_copy(
                rows[buf].at[pl.ds(0, ln)],
                out_ref.at[pl.ds(row0 + off, ln), pl.ds(c * HALF, HALF)],
                ssem[buf])

        def d_out_wait(i, buf):
            off, ln = _drain_chunks[i]
            pltpu.make_async_copy(
                rows[buf].at[pl.ds(0, ln)],
                out_ref.at[pl.ds(row0 + off, ln), pl.ds(c * HALF, HALF)],
                ssem[buf]).wait()

        d_in(0, 0)
        for i in range(n):
            buf = i % 2
            other = 1 - buf
            d_in_wait(i, buf)
            if i + 1 < n:
                if i >= 1:
                    d_out_wait(i - 1, other)
                d_in(i + 1, other)
            d_out(i, buf)
        d_out_wait(n - 2, (n - 2) % 2)
        d_out_wait(n - 1, (n - 1) % 2)

    def _do_pass(src_e, dst_e, feat2, count_flag):
        def _gidx_compute(st):
            srcb, _, gidxb = idxset[st]
            for j in range(SCG):
                for q in range(GSZ // 16):
                    gidxb[j, pl.ds(q * 16, 16)] = (
                        srcb[j, pl.ds(q * 16, 16)] * 2 + c)

        def load_chunk(k, st):
            srcb, dstb, _ = idxset[st]
            r0 = s * NG + k * SCG
            pltpu.sync_copy(src_e.at[pl.ds(r0, SCG)], srcb)
            pltpu.sync_copy(dst_e.at[pl.ds(r0, SCG)], dstb)
            _gidx_compute(st)

        def load_chunk_start(k, st):
            srcb, dstb, _ = idxset[st]
            r0 = s * NG + k * SCG
            pltpu.async_copy(src_e.at[pl.ds(r0, SCG)], srcb, isem[st])
            pltpu.async_copy(dst_e.at[pl.ds(r0, SCG)], dstb, isem[st])

        def load_chunk_finish(st):
            srcb, dstb, _ = idxset[st]
            pltpu.make_async_copy(src_e.at[pl.ds(0, SCG)], srcb,
                                  isem[st]).wait()
            pltpu.make_async_copy(dst_e.at[pl.ds(0, SCG)], dstb,
                                  isem[st]).wait()
            _gidx_compute(st)

        def start_gather(j, st, buf):
            gidxb = idxset[st][2]
            pltpu.async_copy(feat2.at[gidxb.at[j]], rows[buf], gsem[buf])

        def wait_gather(buf):
            pltpu.make_async_copy(feat2.at[gidxA.at[0]], rows[buf],
                                  gsem[buf]).wait()

        def start_scatter(j, st, buf):
            dstb = idxset[st][1]
            pltpu.async_copy(rows[buf], accum.at[dstb.at[j]], ssem[buf],
                             add=True)

        def wait_scatter(buf):
            pltpu.make_async_copy(rows[buf], accum.at[dstA.at[0]],
                                  ssem[buf]).wait()

        def hist_group(j, st):
            dstb = idxset[st][1]
            for q in range(GSZ // 16):
                dv = dstb[j, pl.ds(q * 16, 16)]
                plsc.addupdate_scatter(histb, [dv], ones16)

        def sixteen_groups(t, first, last):
            for j in range(16):
                st = j // 8
                buf = j % 2
                other = 1 - buf
                if not (first and j == 0):
                    wait_scatter(other)
                if j < 15:
                    start_gather((j + 1) % 8, (j + 1) // 8, other)
                else:
                    if last is None:
                        @pl.when(t < NSC // 2 - 1)
                        def _():
                            start_gather(0, 0, other)
                    elif not last:
                        start_gather(0, 0, other)
                wait_gather(buf)
                start_scatter(j % 8, st, buf)

                @pl.when(count_flag)
                def _():
                    hist_group(j % 8, st)

                if j == 1:
                    load_chunk_start(2 * t + 1, 1)
                if j == 5:
                    load_chunk_finish(1)
                if j == 9:
                    load_chunk_start(2 * t + 2, 0)
                if j == 13:
                    load_chunk_finish(0)

        load_chunk(0, 0)
        start_gather(0, 0, 0)
        sixteen_groups(0, True, NSC // 2 == 1)

        def body(t, carry):
            sixteen_groups(t, False, None)
            return carry
        lax.fori_loop(1, NSC // 2, body, None)

        wait_scatter(1)

    _zero_accum()
    plsc.subcore_barrier()

    _do_pass(src_h, dst_h, hbar2, c == 0)
    plsc.subcore_barrier()
    _drain(sum_h)
    _zero_accum()
    plsc.subcore_barrier()

    _do_pass(src_s, dst_s, hdelta2, c == 1)
    plsc.subcore_barrier()
    _drain(sum_s)

    def _reduce_counts(out):
        pltpu.sync_copy(histb, hist_part.at[s])
        plsc.subcore_barrier()
        nblk = jnp.where(s == NS - 1, NPAD // HALF - 5 * (NS - 1), 5)

        def blk(b, carry):
            col0 = pl.multiple_of((s * 5 + b) * HALF, HALF)
            pltpu.sync_copy(hist_part.at[:, pl.ds(col0, HALF)], redb)
            for q in range(HALF // 16):
                acc = zero16
                for j in range(NS):
                    acc = acc + redb[j, pl.ds(q * 16, 16)]
                rowsA[0, pl.ds(q * 16, 16)] = acc
            pltpu.sync_copy(rowsA.at[0], out.at[pl.ds(col0, HALF)])
            return carry
        lax.fori_loop(0, nblk, blk, None)

    @pl.when(c == 0)
    def _():
        _reduce_counts(cnt_h)

    @pl.when(c == 1)
    def _():
        _reduce_counts(cnt_s)


@functools.cache
def _get_sc_agg(interpret: bool = False):
    mesh = plsc.VectorSubcoreMesh(
        core_axis_name="c", subcore_axis_name="s", num_cores=NC, num_subcores=NS
    )
    return pl.kernel(
        _sc_agg_body,
        out_type=(
            jax.ShapeDtypeStruct((NPAD, D), jnp.float32),
            jax.ShapeDtypeStruct((NPAD, D), jnp.float32),
            jax.ShapeDtypeStruct((NPAD,), jnp.float32),
            jax.ShapeDtypeStruct((NPAD,), jnp.float32),
        ),
        mesh=mesh,
        scratch_types=[
            pltpu.VMEM_SHARED((NPAD, HALF), jnp.float32),
            pltpu.VMEM_SHARED((NS, NPAD), jnp.float32),
            pltpu.VMEM((GSZ, HALF), jnp.float32),
            pltpu.VMEM((GSZ, HALF), jnp.float32),
            pltpu.VMEM((SCG, GSZ), jnp.int32),
            pltpu.VMEM((SCG, GSZ), jnp.int32),
            pltpu.VMEM((SCG, GSZ), jnp.int32),
            pltpu.VMEM((SCG, GSZ), jnp.int32),
            pltpu.VMEM((SCG, GSZ), jnp.int32),
            pltpu.VMEM((SCG, GSZ), jnp.int32),
            pltpu.VMEM((NPAD,), jnp.float32),
            pltpu.VMEM((NS, HALF), jnp.float32),
            pltpu.SemaphoreType.DMA,
            pltpu.SemaphoreType.DMA,
            pltpu.SemaphoreType.DMA,
            pltpu.SemaphoreType.DMA,
            pltpu.SemaphoreType.DMA,
            pltpu.SemaphoreType.DMA,
        ],
        compiler_params=pltpu.CompilerParams(needs_layout_passes=False),
        interpret=interpret,
    )


_TCR = 1000


def _tc_pre_body(h_ref, hb_ref, w1_ref, b_ref, p_ref, hd_ref):
    hv = h_ref[...]
    hd_ref[...] = hv - hb_ref[...]
    p_ref[...] = (
        jnp.dot(hv, w1_ref[...], preferred_element_type=jnp.float32) + b_ref[...]
    )


def _tc_post_body(p_ref, sh0_ref, sh1_ref, ss0_ref, ss1_ref, ch_ref, cs_ref,
                  w2a_ref, w2b_ref, o_ref):
    rh = 1.0 / jnp.maximum(ch_ref[...], 1.0)
    rs = 1.0 / jnp.maximum(cs_ref[...], 1.0)
    hn0 = sh0_ref[...] * rh + ss0_ref[...] * rs
    hn1 = sh1_ref[...] * rh + ss1_ref[...] * rs
    acc = (
        p_ref[...]
        + jnp.dot(hn0, w2a_ref[...], preferred_element_type=jnp.float32)
        + jnp.dot(hn1, w2b_ref[...], preferred_element_type=jnp.float32)
    )
    o_ref[...] = jnp.maximum(acc, 0.0)


def _tc_pre(H_l, HBar_l, W1, b2):
    return pl.pallas_call(
        _tc_pre_body,
        grid=(N // _TCR,),
        in_specs=[
            pl.BlockSpec((_TCR, D), lambda i: (i, 0)),
            pl.BlockSpec((_TCR, D), lambda i: (i, 0)),
            pl.BlockSpec((D, D), lambda i: (0, 0)),
            pl.BlockSpec((1, D), lambda i: (0, 0)),
        ],
        out_specs=[
            pl.BlockSpec((_TCR, D), lambda i: (i, 0)),
            pl.BlockSpec((_TCR, D), lambda i: (i, 0)),
        ],
        out_shape=[
            jax.ShapeDtypeStruct((N, D), jnp.float32),
            jax.ShapeDtypeStruct((N, D), jnp.float32),
        ],
    )(H_l, HBar_l, W1, b2)


def _tc_post(P, sum_h, sum_s, cnt_h2, cnt_s2, W2a, W2b):
    return pl.pallas_call(
        _tc_post_body,
        grid=(N // _TCR,),
        in_specs=[
            pl.BlockSpec((_TCR, D), lambda i: (i, 0)),
            pl.BlockSpec((_TCR, HALF), lambda i: (i, 0)),
            pl.BlockSpec((_TCR, HALF), lambda i: (i, 1)),
            pl.BlockSpec((_TCR, HALF), lambda i: (i, 0)),
            pl.BlockSpec((_TCR, HALF), lambda i: (i, 1)),
            pl.BlockSpec((_TCR, 1), lambda i: (i, 0)),
            pl.BlockSpec((_TCR, 1), lambda i: (i, 0)),
            pl.BlockSpec((HALF, D), lambda i: (0, 0)),
            pl.BlockSpec((HALF, D), lambda i: (0, 0)),
        ],
        out_specs=pl.BlockSpec((_TCR, D), lambda i: (i, 0)),
        out_shape=jax.ShapeDtypeStruct((N, D), jnp.float32),
    )(P, sum_h, sum_h, sum_s, sum_s, cnt_h2, cnt_s2, W2a, W2b)


def kernel(H_l, HBar_l, edge_index_history, edge_index_sampled, W, b):
    src_h, dst_h = edge_index_history[0], edge_index_history[1]
    src_s, dst_s = edge_index_sampled[0], edge_index_sampled[1]

    pad = EPAD - E
    pad_src = jnp.zeros((pad,), jnp.int32)
    pad_dst = jnp.full((pad,), PAD_DST, jnp.int32)
    src_h = jnp.concatenate([src_h, pad_src]).reshape(EROWS, GSZ)
    dst_h = jnp.concatenate([dst_h, pad_dst]).reshape(EROWS, GSZ)
    src_s = jnp.concatenate([src_s, pad_src]).reshape(EROWS, GSZ)
    dst_s = jnp.concatenate([dst_s, pad_dst]).reshape(EROWS, GSZ)

    P, hdelta = _tc_pre(H_l, HBar_l, W[:D], b.reshape(1, D))

    hbar2 = HBar_l.reshape(2 * N, HALF)
    hdelta2 = hdelta.reshape(2 * N, HALF)

    sum_h, sum_s, cnt_h, cnt_s = _get_sc_agg()(
        hbar2, hdelta2, src_h, dst_h, src_s, dst_s
    )

    return _tc_post(
        P, sum_h, sum_s,
        cnt_h[:N].reshape(N, 1), cnt_s[:N].reshape(N, 1),
        W[D:D + HALF], W[D + HALF:],
    )
